# baseline (speedup 1.0000x reference)
import functools

import jax
import jax.numpy as jnp
from jax import lax
from jax.experimental import pallas as pl
from jax.experimental.pallas import tpu as pltpu
from jax.experimental.pallas import tpu_sc as plsc

N = 10000
E = 320000
NC = 2
NS = 16
NW = NC * NS
CE = E // NW
CB = 80
NCHUNK = CE // CB
NPT = N // NS
EP = 2000
NEP = CE // EP
SPP = EP // 16

_MESH = plsc.VectorSubcoreMesh(
    core_axis_name="c", subcore_axis_name="s", num_cores=NC, num_subcores=NS)

_SC_PARAMS = pltpu.CompilerParams(needs_layout_passes=False)

_HI = jax.lax.Precision.HIGHEST


def _gat_feat_body(x_ref, w_ref, al_ref, ar_ref, feat_ref, el_ref, er_ref):
    feat = jnp.dot(x_ref[...].astype(jnp.bfloat16),
                   w_ref[...].astype(jnp.bfloat16),
                   preferred_element_type=jnp.float32)
    els = []
    ers = []
    for h in range(3):
        fh = feat[:, 128 * h:128 * (h + 1)]
        feat_ref[h] = fh
        els.append(jnp.sum(fh * al_ref[h][None, :], axis=1, keepdims=True))
        ers.append(jnp.sum(fh * ar_ref[h][None, :], axis=1, keepdims=True))
    el_ref[...] = jnp.concatenate(els, axis=1)
    er_ref[...] = jnp.concatenate(ers, axis=1)


def _gat_feat(x, Wp, alp, arp):
    bn = 2000
    f = x.shape[1]
    grid = (N // bn,)
    return pl.pallas_call(
        _gat_feat_body,
        grid=grid,
        in_specs=[
            pl.BlockSpec((bn, f), lambda i: (i, 0)),
            pl.BlockSpec((f, 384), lambda i: (0, 0)),
            pl.BlockSpec((3, 128), lambda i: (0, 0)),
            pl.BlockSpec((3, 128), lambda i: (0, 0)),
        ],
        out_specs=[
            pl.BlockSpec((3, bn, 128), lambda i: (0, i, 0)),
            pl.BlockSpec((bn, 3), lambda i: (i, 0)),
            pl.BlockSpec((bn, 3), lambda i: (i, 0)),
        ],
        out_shape=[
            jax.ShapeDtypeStruct((3, N, 128), jnp.float32),
            jax.ShapeDtypeStruct((N, 3), jnp.float32),
            jax.ShapeDtypeStruct((N, 3), jnp.float32),
        ],
    )(x, Wp, alp, arp)


def _prep_body(sp_ref, dp_ref, s_ref, dfac_ref):
    s = jnp.sum(sp_ref[...], axis=0)
    s_ref[...] = jnp.maximum(s, 1e-30)
    d = jnp.sum(dp_ref[...], axis=0)
    dfac_ref[...] = jnp.maximum(d, 1.0) ** -0.5


def _prep(spart, dpart):
    return pl.pallas_call(
        _prep_body,
        out_shape=[
            jax.ShapeDtypeStruct((3, N), jnp.float32),
            jax.ShapeDtypeStruct((2, N), jnp.float32),
        ],
    )(spart, dpart)


def _mid1_body(relu_flag, bn, rp_ref, dfac_ref, bp_ref, w_ref, o_ref):
    rs = []
    for h in range(3):
        r = rp_ref[h, 0] + rp_ref[h, 1]
        rs.append(r + bp_ref[h][None, :])
    x = jnp.maximum(jnp.maximum(rs[0], rs[1]), rs[2])
    if relu_flag:
        x = jnp.maximum(x, 0.0)
    xh = x * dfac_ref[:, 0][:, None]
    o_ref[...] = jnp.dot(xh.astype(jnp.bfloat16),
                         w_ref[...].astype(jnp.bfloat16),
                         preferred_element_type=jnp.float32)


def _mid1(rst_part, dfac, bp, Wg, relu_flag):
    bn = 2000
    return pl.pallas_call(
        functools.partial(_mid1_body, relu_flag, bn),
        grid=(N // bn,),
        in_specs=[
            pl.BlockSpec((3, NC, bn, 128), lambda i: (0, 0, i, 0)),
            pl.BlockSpec((bn, 2), lambda i: (i, 0)),
            pl.BlockSpec((3, 128), lambda i: (0, 0)),
            pl.BlockSpec((128, 128), lambda i: (0, 0)),
        ],
        out_specs=pl.BlockSpec((bn, 128), lambda i: (i, 0)),
        out_shape=jax.ShapeDtypeStruct((N, 128), jnp.float32),
    )(rst_part, dfac, bp, Wg)


def _mid2_body(bn, ap_ref, dfac_ref, b_ref, w_ref, o_ref):
    y = ((ap_ref[0] + ap_ref[1]) * dfac_ref[:, 1][:, None]
         + b_ref[0][None, :])
    y = jnp.maximum(y, 0.0)
    yh = y * dfac_ref[:, 0][:, None]
    o_ref[...] = jnp.dot(yh.astype(jnp.bfloat16),
                         w_ref[...].astype(jnp.bfloat16),
                         preferred_element_type=jnp.float32)


def _mid2(agg_part, dfac, b, W2):
    bn = 2000
    return pl.pallas_call(
        functools.partial(_mid2_body, bn),
        grid=(N // bn,),
        in_specs=[
            pl.BlockSpec((NC, bn, 128), lambda i: (0, i, 0)),
            pl.BlockSpec((bn, 2), lambda i: (i, 0)),
            pl.BlockSpec((1, 128), lambda i: (0, 0)),
            pl.BlockSpec((128, 128), lambda i: (0, 0)),
        ],
        out_specs=pl.BlockSpec((bn, 128), lambda i: (i, 0)),
        out_shape=jax.ShapeDtypeStruct((N, 128), jnp.float32),
    )(agg_part, dfac, b.reshape(1, 128), W2)


def _fin_body(bn, ap_ref, dfac_ref, b_ref, o_ref):
    y = ((ap_ref[0] + ap_ref[1]) * dfac_ref[:, 1][:, None]
         + b_ref[0][None, :])
    o_ref[...] = jnp.maximum(y, 0.0)


def _fin(agg_part, dfac, b):
    bn = 2000
    return pl.pallas_call(
        functools.partial(_fin_body, bn),
        grid=(N // bn,),
        in_specs=[
            pl.BlockSpec((NC, bn, 128), lambda i: (0, i, 0)),
            pl.BlockSpec((bn, 2), lambda i: (i, 0)),
            pl.BlockSpec((1, 128), lambda i: (0, 0)),
        ],
        out_specs=pl.BlockSpec((bn, 128), lambda i: (i, 0)),
        out_shape=jax.ShapeDtypeStruct((N, 128), jnp.float32),
    )(agg_part, dfac, b.reshape(1, 128))


def _heads_body(x1_ref, x2_ref, x3_ref, r1w_ref, r1b_ref, r2w_ref, r2b_ref,
                clw_ref, clb_ref, reg1_ref, reg2_ref, cls_ref, hg3_ref, w_ref):
    hg1 = jnp.sum(x1_ref[...], axis=0, keepdims=True) / N
    hg2 = jnp.sum(x2_ref[...], axis=0, keepdims=True) / N
    hg3 = jnp.sum(x3_ref[...], axis=0, keepdims=True) / N
    reg1_ref[...] = jnp.dot(hg1, r1w_ref[...], preferred_element_type=jnp.float32,
                            precision=_HI) + r1b_ref[...]
    reg2_ref[...] = jnp.dot(hg2, r2w_ref[...], preferred_element_type=jnp.float32,
                            precision=_HI) + r2b_ref[...]
    cls_ref[...] = jnp.dot(hg3, clw_ref[...], preferred_element_type=jnp.float32,
                           precision=_HI) + clb_ref[...]
    hg3_ref[...] = hg3
    w_ref[...] = x3_ref[:, 78:79]


def _heads(x1f, x2f, x3f, r1_W, r1_b, r2_W, r2_b, cl_W, cl_b):
    return pl.pallas_call(
        _heads_body,
        out_shape=[
            jax.ShapeDtypeStruct((1, 1), jnp.float32),
            jax.ShapeDtypeStruct((1, 1), jnp.float32),
            jax.ShapeDtypeStruct((1, 10), jnp.float32),
            jax.ShapeDtypeStruct((1, 128), jnp.float32),
            jax.ShapeDtypeStruct((N, 1), jnp.float32),
        ],
    )(x1f, x2f, x3f, r1_W, r1_b.reshape(1, 1), r2_W, r2_b.reshape(1, 1),
      cl_W, cl_b.reshape(1, 10))


def _stats_body(srcf_hbm, dstf_hbm, el_hbm, er_hbm, zer_hbm,
                ex_hbm, spart_hbm,
                src_v, dst_v, el_v, er_v, sp_v, exb_v):
    c = lax.axis_index("c")
    s = lax.axis_index("s")
    w = s * NC + c

    pltpu.sync_copy(srcf_hbm.at[w], src_v)
    pltpu.sync_copy(dstf_hbm.at[w], dst_v)
    pltpu.sync_copy(el_hbm, el_v)
    pltpu.sync_copy(er_hbm, er_v)
    pltpu.sync_copy(zer_hbm, sp_v)

    z16 = jnp.zeros((16,), jnp.int32)
    for p in range(NEP):
        def _step(k, _):
            j = p * EP + k * 16
            s16 = src_v[0, pl.ds(j, 16)] * 3
            d16 = dst_v[0, pl.ds(j, 16)]
            d3 = d16 * 3
            for h in range(3):
                e = (plsc.load_gather(el_v, [z16, s16 + h])
                     + plsc.load_gather(er_v, [z16, d3 + h]))
                e = jnp.maximum(e, 0.2 * e)
                exh = jnp.exp(e)
                exb_v[h, pl.ds(k * 16, 16)] = exh
                plsc.addupdate_scatter(sp_v, [z16, d16 + h * N], exh)
            return 0
        lax.fori_loop(0, EP // 16, _step, 0)
        pltpu.sync_copy(exb_v, ex_hbm.at[w, p])

    pltpu.sync_copy(sp_v, spart_hbm.at[w])


def _stats_sc(srcf, dstf, el, er, zeros_n3):
    kfn = pl.kernel(
        _stats_body,
        out_type=[
            jax.ShapeDtypeStruct((NW, NEP, 3, EP), jnp.float32),
            jax.ShapeDtypeStruct((NW, 1, 3 * N), jnp.float32),
        ],
        mesh=_MESH,
        compiler_params=_SC_PARAMS,
        scratch_types=[
            pltpu.VMEM((1, CE), jnp.int32),
            pltpu.VMEM((1, CE), jnp.int32),
            pltpu.VMEM((1, 3 * N), jnp.float32),
            pltpu.VMEM((1, 3 * N), jnp.float32),
            pltpu.VMEM((1, 3 * N), jnp.float32),
            pltpu.VMEM((3, EP), jnp.float32),
        ],
    )
    return kfn(srcf, dstf, el, er, zeros_n3)


def _deg_body(srcf_hbm, dstf_hbm, zer_hbm, dpart_hbm, src_v, dst_v, dp_v):
    c = lax.axis_index("c")
    s = lax.axis_index("s")
    w = s * NC + c

    pltpu.sync_copy(srcf_hbm.at[w], src_v)
    pltpu.sync_copy(dstf_hbm.at[w], dst_v)
    pltpu.sync_copy(zer_hbm, dp_v)

    ones16 = jnp.ones((16,), jnp.float32)
    z16 = jnp.zeros((16,), jnp.int32)

    def _step(k, _):
        s16 = src_v[0, pl.ds(k * 16, 16)]
        d16 = dst_v[0, pl.ds(k * 16, 16)]
        plsc.addupdate_scatter(dp_v, [z16, s16], ones16)
        plsc.addupdate_scatter(dp_v, [z16, d16 + N], ones16)
        return 0
    lax.fori_loop(0, CE // 16, _step, 0)

    pltpu.sync_copy(dp_v, dpart_hbm.at[w])


def _deg_sc(srcf, dstf, zeros_n2):
    kfn = pl.kernel(
        _deg_body,
        out_type=jax.ShapeDtypeStruct((NW, 1, 2 * N), jnp.float32),
        mesh=_MESH,
        compiler_params=_SC_PARAMS,
        scratch_types=[
            pltpu.VMEM((1, CE), jnp.int32),
            pltpu.VMEM((1, CE), jnp.int32),
            pltpu.VMEM((1, 2 * N), jnp.float32),
        ],
    )
    return kfn(srcf, dstf, zeros_n2)


def _gatagg_body(srcf_hbm, dstf_hbm, featp_hbm, ex_hbm,
                 zer_hbm, rst_hbm,
                 idx_v, dst_v, ex3_v, rbuf0, rbuf1, acc, sem0, sem1):
    c = lax.axis_index("c")
    s = lax.axis_index("s")
    w = s * NC + c

    pltpu.sync_copy(dstf_hbm.at[w], dst_v)
    pltpu.sync_copy(srcf_hbm.at[w], idx_v)

    def _s16(k):
        return idx_v[0, pl.ds(k * 16, 16)]

    def _d16(k):
        return dst_v[0, pl.ds(k * 16, 16)]

    for h in range(3):
        if h > 0:
            def _mkidx(k, _):
                idx_v[0, pl.ds(k * 16, 16)] = idx_v[0, pl.ds(k * 16, 16)] + N
                return 0
            lax.fori_loop(0, CE // 16, _mkidx, 0)

        pltpu.sync_copy(zer_hbm, acc.at[pl.ds(s * NPT, NPT)])
        plsc.subcore_barrier()

        for p in range(NEP):
            pltpu.sync_copy(ex_hbm.at[w, p], ex3_v)
            base = p * SPP

            def _scale(buf, kl):
                def _edge(j, _):
                    ev = plsc.load_gather(
                        ex3_v,
                        [jnp.full((16,), h, jnp.int32),
                         jnp.broadcast_to(kl * 16 + j, (16,)).astype(jnp.int32)])
                    for v in range(8):
                        buf[j, pl.ds(v * 16, 16)] = buf[j, pl.ds(v * 16, 16)] * ev
                    return 0
                lax.fori_loop(0, 16, _edge, 0)

            pltpu.async_copy(featp_hbm.at[_s16(base)], rbuf0, sem0)

            def _dstep(g, _):
                k0 = base + 2 * g
                pltpu.make_async_copy(featp_hbm.at[_s16(k0)], rbuf0, sem0).wait()
                d1 = pltpu.async_copy(featp_hbm.at[_s16(k0 + 1)], rbuf1, sem1)
                _scale(rbuf0, 2 * g)
                pltpu.sync_copy(rbuf0, acc.at[_d16(k0)], add=True)
                d1.wait()
                pltpu.async_copy(featp_hbm.at[_s16(k0 + 2)], rbuf0, sem0)
                _scale(rbuf1, 2 * g + 1)
                pltpu.sync_copy(rbuf1, acc.at[_d16(k0 + 1)], add=True)
                return 0
            lax.fori_loop(0, SPP // 2, _dstep, 0)
            kt = base + SPP - 1
            pltpu.make_async_copy(featp_hbm.at[_s16(kt)], rbuf0, sem0).wait()
            _scale(rbuf0, SPP - 1)
            pltpu.sync_copy(rbuf0, acc.at[_d16(kt)], add=True)

        plsc.subcore_barrier()
        pltpu.sync_copy(acc.at[pl.ds(s * NPT, NPT)], rst_hbm.at[h, c, s])


def _gatagg_sc(srcf, dstf, featp, ex, zeros_npt):
    kfn = pl.kernel(
        _gatagg_body,
        out_type=jax.ShapeDtypeStruct((3, NC, NS, NPT, 128), jnp.float32),
        mesh=_MESH,
        compiler_params=_SC_PARAMS,
        scratch_types=[
            pltpu.VMEM((1, CE), jnp.int32),
            pltpu.VMEM((1, CE), jnp.int32),
            pltpu.VMEM((3, EP), jnp.float32),
            pltpu.VMEM((16, 128), jnp.float32),
            pltpu.VMEM((16, 128), jnp.float32),
            pltpu.VMEM_SHARED((N, 128), jnp.float32),
            pltpu.SemaphoreType.DMA,
            pltpu.SemaphoreType.DMA,
        ],
    )
    return kfn(srcf, dstf, featp, ex, zeros_npt)


def _att_body(dstf_hbm, ex_hbm, s_hbm, a_hbm, att_hbm,
              dst_v, s_v, ex3_v, ab_v, att_v):
    c = lax.axis_index("c")
    s = lax.axis_index("s")
    w = s * NC + c

    pltpu.sync_copy(dstf_hbm.at[w], dst_v)
    pltpu.sync_copy(s_hbm, s_v)

    z16 = jnp.zeros((16,), jnp.int32)
    for p in range(NEP):
        pltpu.sync_copy(ex_hbm.at[w, p], ex3_v)

        def _att(k, _):
            j = p * EP + k * 16
            d16 = dst_v[0, pl.ds(j, 16)]
            amax = None
            for h2 in range(3):
                a16 = (ex3_v[h2, pl.ds(k * 16, 16)]
                       / plsc.load_gather(s_v, [z16, d16 + h2 * N]))
                ab_v[h2, pl.ds(k * 16, 16)] = a16
                amax = a16 if amax is None else jnp.maximum(amax, a16)
            att_v[0, pl.ds(j, 16)] = amax
            return 0
        lax.fori_loop(0, EP // 16, _att, 0)
        pltpu.sync_copy(ab_v, a_hbm.at[w, p])

    pltpu.sync_copy(att_v, att_hbm.at[w])


def _att_sc(dstf, ex, s_tot):
    kfn = pl.kernel(
        _att_body,
        out_type=[
            jax.ShapeDtypeStruct((NW, NEP, 3, EP), jnp.float32),
            jax.ShapeDtypeStruct((NW, 1, CE), jnp.float32),
        ],
        mesh=_MESH,
        compiler_params=_SC_PARAMS,
        scratch_types=[
            pltpu.VMEM((1, CE), jnp.int32),
            pltpu.VMEM((1, 3 * N), jnp.float32),
            pltpu.VMEM((3, EP), jnp.float32),
            pltpu.VMEM((3, EP), jnp.float32),
            pltpu.VMEM((1, CE), jnp.float32),
        ],
    )
    return kfn(dstf, ex, s_tot)


def _gcnagg_body(srcf_hbm, dstf_hbm, tbl_hbm, zer_hbm, out_hbm,
                 src_v, dst_v, rbuf0, rbuf1, acc, sem0, sem1):
    c = lax.axis_index("c")
    s = lax.axis_index("s")
    w = s * NC + c

    pltpu.sync_copy(srcf_hbm.at[w], src_v)
    pltpu.sync_copy(dstf_hbm.at[w], dst_v)
    pltpu.sync_copy(zer_hbm, acc.at[pl.ds(s * NPT, NPT)])
    plsc.subcore_barrier()

    def _s16(k):
        return src_v[0, pl.ds(k * 16, 16)]

    def _d16(k):
        return dst_v[0, pl.ds(k * 16, 16)]

    nstep = CE // 16
    pltpu.async_copy(tbl_hbm.at[_s16(0)], rbuf0, sem0)

    def _dstep(g, _):
        k0 = 2 * g
        pltpu.make_async_copy(tbl_hbm.at[_s16(k0)], rbuf0, sem0).wait()
        d1 = pltpu.async_copy(tbl_hbm.at[_s16(k0 + 1)], rbuf1, sem1)
        pltpu.sync_copy(rbuf0, acc.at[_d16(k0)], add=True)
        d1.wait()
        pltpu.async_copy(tbl_hbm.at[_s16(k0 + 2)], rbuf0, sem0)
        pltpu.sync_copy(rbuf1, acc.at[_d16(k0 + 1)], add=True)
        return 0
    lax.fori_loop(0, nstep // 2, _dstep, 0)
    kt = nstep - 1
    pltpu.make_async_copy(tbl_hbm.at[_s16(kt)], rbuf0, sem0).wait()
    pltpu.sync_copy(rbuf0, acc.at[_d16(kt)], add=True)

    plsc.subcore_barrier()
    pltpu.sync_copy(acc.at[pl.ds(s * NPT, NPT)], out_hbm.at[c, s])


def _gcnagg_sc(srcf, dstf, tbl, zeros_npt):
    kfn = pl.kernel(
        _gcnagg_body,
        out_type=jax.ShapeDtypeStruct((NC, NS, NPT, 128), jnp.float32),
        mesh=_MESH,
        compiler_params=_SC_PARAMS,
        scratch_types=[
            pltpu.VMEM((1, CE), jnp.int32),
            pltpu.VMEM((1, CE), jnp.int32),
            pltpu.VMEM((16, 128), jnp.float32),
            pltpu.VMEM((16, 128), jnp.float32),
            pltpu.VMEM_SHARED((N, 128), jnp.float32),
            pltpu.SemaphoreType.DMA,
            pltpu.SemaphoreType.DMA,
        ],
    )
    return kfn(srcf, dstf, tbl, zeros_npt)


def _pad_gat(W, al, ar, f):
    Wp = W.reshape(W.shape[0], 3, f)
    alp, arp = al, ar
    if f < 128:
        Wp = jnp.pad(Wp, ((0, 0), (0, 0), (0, 128 - f)))
        alp = jnp.pad(al, ((0, 0), (0, 128 - f)))
        arp = jnp.pad(ar, ((0, 0), (0, 128 - f)))
    return Wp.reshape(W.shape[0], 384), alp, arp


def _pad_b(b, f):
    bp = b.reshape(3, f)
    if f < 128:
        bp = jnp.pad(bp, ((0, 0), (0, 128 - f)))
    return bp


def _pad_gcn_w(W):
    if W.shape[0] == 128:
        return W
    return jnp.pad(W, ((0, 128 - W.shape[0]), (0, 0)))


def _graph_pipeline(src, dst, x, Wp, alp, arp, bp, f, gcnA_W, gcnA_b,
                    gcnB_W, gcnB_b, relu_after_max, zn3, zn2, znpt):
    srcf = src.reshape(NW, 1, CE)
    dstf = dst.reshape(NW, 1, CE)

    featp, el, er = _gat_feat(x, Wp, alp, arp)
    ex, spart = _stats_sc(srcf, dstf, el.reshape(1, 3 * N),
                          er.reshape(1, 3 * N), zn3)
    dpart = _deg_sc(srcf, dstf, zn2)
    s_tot, dfac = _prep(spart.reshape(NW, 3, N), dpart.reshape(NW, 2, N))
    a_planes, att = _att_sc(dstf, ex, s_tot.reshape(1, 3 * N))
    rst_part = _gatagg_sc(srcf, dstf, featp.reshape(3 * N, 128), a_planes, znpt)
    rst_part = rst_part.reshape(3, NC, N, 128)

    dfac_t = dfac.T
    y1 = _mid1(rst_part, dfac_t, bp, gcnA_W, relu_after_max)
    agg1 = _gcnagg_sc(srcf, dstf, y1, znpt).reshape(NC, N, 128)
    y2 = _mid2(agg1, dfac_t, gcnA_b, gcnB_W)
    agg2 = _gcnagg_sc(srcf, dstf, y2, znpt).reshape(NC, N, 128)
    xf = _fin(agg2, dfac_t, gcnB_b)
    return xf, att


def kernel(g1, g2, g3, h1, h2, h3, gat1_W, gat1_al, gat1_ar, gat1_b,
           gat2_W, gat2_al, gat2_ar, gat2_b, c1_W, c1_b, c2_W, c2_b,
           c3_W, c3_b, r1_W, r1_b, r2_W, r2_b, cl_W, cl_b):
    zn3 = jnp.zeros((1, 3 * N), jnp.float32)
    zn2 = jnp.zeros((1, 2 * N), jnp.float32)
    znpt = jnp.zeros((NPT, 128), jnp.float32)

    W1p, al1p, ar1p = _pad_gat(gat1_W, gat1_al, gat1_ar, 127)
    b1p = _pad_b(gat1_b, 127)
    W2p, al2p, ar2p = _pad_gat(gat2_W, gat2_al, gat2_ar, 128)
    b2p = _pad_b(gat2_b, 128)
    c3_Wp = _pad_gcn_w(c3_W)

    x1f, att1 = _graph_pipeline(g1[0], g1[1], h1, W1p, al1p, ar1p, b1p, 127,
                                c3_Wp, c3_b, c2_W, c2_b, True, zn3, zn2, znpt)
    x2f, _ = _graph_pipeline(g2[0], g2[1], h2, W1p, al1p, ar1p, b1p, 127,
                             c3_Wp, c3_b, c2_W, c2_b, False, zn3, zn2, znpt)
    x3f, att3 = _graph_pipeline(g3[0], g3[1], h3, W2p, al2p, ar2p, b2p, 128,
                                c1_W, c1_b, c2_W, c2_b, False, zn3, zn2, znpt)

    reg1, reg2, cls, hg3, wcol = _heads(x1f, x2f, x3f, r1_W, r1_b, r2_W, r2_b,
                                        cl_W, cl_b)
    return (reg1, reg2, cls, att1.reshape(E, 1), att3.reshape(E, 1),
            wcol.reshape(N), hg3)


# speedup vs baseline: 1.9093x; 1.9093x over previous
import functools

import jax
import jax.numpy as jnp
from jax import lax
from jax.experimental import pallas as pl
from jax.experimental.pallas import tpu as pltpu
from jax.experimental.pallas import tpu_sc as plsc

N = 10000
E = 320000
NC = 2
NS = 16
NW = NC * NS
CE = E // NW
CB = 80
NCHUNK = CE // CB
NPT = N // NS
EP = 400
NEP = CE // EP

_MESH = plsc.VectorSubcoreMesh(
    core_axis_name="c", subcore_axis_name="s", num_cores=NC, num_subcores=NS)

_SC_PARAMS = pltpu.CompilerParams(needs_layout_passes=False)

_HI = jax.lax.Precision.HIGHEST


def _gat_feat_body(x_ref, w_ref, al_ref, ar_ref, feat_ref, el_ref, er_ref):
    feat = jnp.dot(x_ref[...].astype(jnp.bfloat16),
                   w_ref[...].astype(jnp.bfloat16),
                   preferred_element_type=jnp.float32)
    els = []
    ers = []
    for h in range(3):
        fh = feat[:, 128 * h:128 * (h + 1)]
        feat_ref[h] = fh
        els.append(jnp.sum(fh * al_ref[h][None, :], axis=1, keepdims=True))
        ers.append(jnp.sum(fh * ar_ref[h][None, :], axis=1, keepdims=True))
    el_ref[...] = jnp.concatenate(els, axis=1)
    er_ref[...] = jnp.concatenate(ers, axis=1)


def _gat_feat(x, Wp, alp, arp):
    bn = 2000
    f = x.shape[1]
    grid = (N // bn,)
    return pl.pallas_call(
        _gat_feat_body,
        grid=grid,
        in_specs=[
            pl.BlockSpec((bn, f), lambda i: (i, 0)),
            pl.BlockSpec((f, 384), lambda i: (0, 0)),
            pl.BlockSpec((3, 128), lambda i: (0, 0)),
            pl.BlockSpec((3, 128), lambda i: (0, 0)),
        ],
        out_specs=[
            pl.BlockSpec((3, bn, 128), lambda i: (0, i, 0)),
            pl.BlockSpec((bn, 3), lambda i: (i, 0)),
            pl.BlockSpec((bn, 3), lambda i: (i, 0)),
        ],
        out_shape=[
            jax.ShapeDtypeStruct((3, N, 128), jnp.float32),
            jax.ShapeDtypeStruct((N, 3), jnp.float32),
            jax.ShapeDtypeStruct((N, 3), jnp.float32),
        ],
    )(x, Wp, alp, arp)


def _prep_body(sp_ref, dp_ref, s_ref, dfac_ref):
    s = jnp.sum(sp_ref[...], axis=0)
    s_ref[...] = jnp.maximum(s, 1e-30)
    d = jnp.sum(dp_ref[...], axis=0)
    dfac_ref[...] = jnp.maximum(d, 1.0) ** -0.5


def _prep(spart, dpart):
    return pl.pallas_call(
        _prep_body,
        out_shape=[
            jax.ShapeDtypeStruct((3, N), jnp.float32),
            jax.ShapeDtypeStruct((2, N), jnp.float32),
        ],
    )(spart, dpart)


def _mid1_body(relu_flag, bn, rp_ref, dfac_ref, bp_ref, w_ref, o_ref):
    rs = []
    for h in range(3):
        r = rp_ref[h, 0] + rp_ref[h, 1]
        rs.append(r + bp_ref[h][None, :])
    x = jnp.maximum(jnp.maximum(rs[0], rs[1]), rs[2])
    if relu_flag:
        x = jnp.maximum(x, 0.0)
    xh = x * dfac_ref[:, 0][:, None]
    o_ref[...] = jnp.dot(xh.astype(jnp.bfloat16),
                         w_ref[...].astype(jnp.bfloat16),
                         preferred_element_type=jnp.float32)


def _mid1(rst_part, dfac, bp, Wg, relu_flag):
    bn = 2000
    return pl.pallas_call(
        functools.partial(_mid1_body, relu_flag, bn),
        grid=(N // bn,),
        in_specs=[
            pl.BlockSpec((3, NC, bn, 128), lambda i: (0, 0, i, 0)),
            pl.BlockSpec((bn, 2), lambda i: (i, 0)),
            pl.BlockSpec((3, 128), lambda i: (0, 0)),
            pl.BlockSpec((128, 128), lambda i: (0, 0)),
        ],
        out_specs=pl.BlockSpec((bn, 128), lambda i: (i, 0)),
        out_shape=jax.ShapeDtypeStruct((N, 128), jnp.float32),
    )(rst_part, dfac, bp, Wg)


def _mid2_body(bn, ap_ref, dfac_ref, b_ref, w_ref, o_ref):
    y = ((ap_ref[0] + ap_ref[1]) * dfac_ref[:, 1][:, None]
         + b_ref[0][None, :])
    y = jnp.maximum(y, 0.0)
    yh = y * dfac_ref[:, 0][:, None]
    o_ref[...] = jnp.dot(yh.astype(jnp.bfloat16),
                         w_ref[...].astype(jnp.bfloat16),
                         preferred_element_type=jnp.float32)


def _mid2(agg_part, dfac, b, W2):
    bn = 2000
    return pl.pallas_call(
        functools.partial(_mid2_body, bn),
        grid=(N // bn,),
        in_specs=[
            pl.BlockSpec((NC, bn, 128), lambda i: (0, i, 0)),
            pl.BlockSpec((bn, 2), lambda i: (i, 0)),
            pl.BlockSpec((1, 128), lambda i: (0, 0)),
            pl.BlockSpec((128, 128), lambda i: (0, 0)),
        ],
        out_specs=pl.BlockSpec((bn, 128), lambda i: (i, 0)),
        out_shape=jax.ShapeDtypeStruct((N, 128), jnp.float32),
    )(agg_part, dfac, b.reshape(1, 128), W2)


def _fin_body(bn, ap_ref, dfac_ref, b_ref, o_ref):
    y = ((ap_ref[0] + ap_ref[1]) * dfac_ref[:, 1][:, None]
         + b_ref[0][None, :])
    o_ref[...] = jnp.maximum(y, 0.0)


def _fin(agg_part, dfac, b):
    bn = 2000
    return pl.pallas_call(
        functools.partial(_fin_body, bn),
        grid=(N // bn,),
        in_specs=[
            pl.BlockSpec((NC, bn, 128), lambda i: (0, i, 0)),
            pl.BlockSpec((bn, 2), lambda i: (i, 0)),
            pl.BlockSpec((1, 128), lambda i: (0, 0)),
        ],
        out_specs=pl.BlockSpec((bn, 128), lambda i: (i, 0)),
        out_shape=jax.ShapeDtypeStruct((N, 128), jnp.float32),
    )(agg_part, dfac, b.reshape(1, 128))


def _heads_body(x1_ref, x2_ref, x3_ref, r1w_ref, r1b_ref, r2w_ref, r2b_ref,
                clw_ref, clb_ref, reg1_ref, reg2_ref, cls_ref, hg3_ref, w_ref):
    hg1 = jnp.sum(x1_ref[...], axis=0, keepdims=True) / N
    hg2 = jnp.sum(x2_ref[...], axis=0, keepdims=True) / N
    hg3 = jnp.sum(x3_ref[...], axis=0, keepdims=True) / N
    reg1_ref[...] = jnp.dot(hg1, r1w_ref[...], preferred_element_type=jnp.float32,
                            precision=_HI) + r1b_ref[...]
    reg2_ref[...] = jnp.dot(hg2, r2w_ref[...], preferred_element_type=jnp.float32,
                            precision=_HI) + r2b_ref[...]
    cls_ref[...] = jnp.dot(hg3, clw_ref[...], preferred_element_type=jnp.float32,
                           precision=_HI) + clb_ref[...]
    hg3_ref[...] = hg3
    w_ref[...] = x3_ref[:, 78:79]


def _heads(x1f, x2f, x3f, r1_W, r1_b, r2_W, r2_b, cl_W, cl_b):
    return pl.pallas_call(
        _heads_body,
        out_shape=[
            jax.ShapeDtypeStruct((1, 1), jnp.float32),
            jax.ShapeDtypeStruct((1, 1), jnp.float32),
            jax.ShapeDtypeStruct((1, 10), jnp.float32),
            jax.ShapeDtypeStruct((1, 128), jnp.float32),
            jax.ShapeDtypeStruct((N, 1), jnp.float32),
        ],
    )(x1f, x2f, x3f, r1_W, r1_b.reshape(1, 1), r2_W, r2_b.reshape(1, 1),
      cl_W, cl_b.reshape(1, 10))


def _stats_body(srcf_hbm, dstf_hbm, el_hbm, er_hbm, zer_hbm,
                ex_hbm, spart_hbm,
                src_v, dst_v, el_v, er_v, sp_v, exb_v):
    c = lax.axis_index("c")
    s = lax.axis_index("s")
    w = s * NC + c

    pltpu.sync_copy(srcf_hbm.at[w], src_v)
    pltpu.sync_copy(dstf_hbm.at[w], dst_v)
    pltpu.sync_copy(el_hbm, el_v)
    pltpu.sync_copy(er_hbm, er_v)
    pltpu.sync_copy(zer_hbm, sp_v)

    z16 = jnp.zeros((16,), jnp.int32)
    for p in range(NEP):
        def _step(k, _):
            j = p * EP + k * 16
            s16 = src_v[0, pl.ds(j, 16)] * 3
            d16 = dst_v[0, pl.ds(j, 16)]
            d3 = d16 * 3
            for h in range(3):
                e = (plsc.load_gather(el_v, [z16, s16 + h])
                     + plsc.load_gather(er_v, [z16, d3 + h]))
                e = jnp.maximum(e, 0.2 * e)
                exh = jnp.exp(e)
                exb_v[h, pl.ds(k * 16, 16)] = exh
                plsc.addupdate_scatter(sp_v, [z16, d16 + h * N], exh)
            return 0
        lax.fori_loop(0, EP // 16, _step, 0)
        pltpu.sync_copy(exb_v, ex_hbm.at[w, p])

    pltpu.sync_copy(sp_v, spart_hbm.at[w])


def _stats_sc(srcf, dstf, el, er, zeros_n3):
    kfn = pl.kernel(
        _stats_body,
        out_type=[
            jax.ShapeDtypeStruct((NW, NEP, 3, EP), jnp.float32),
            jax.ShapeDtypeStruct((NW, 1, 3 * N), jnp.float32),
        ],
        mesh=_MESH,
        compiler_params=_SC_PARAMS,
        scratch_types=[
            pltpu.VMEM((1, CE), jnp.int32),
            pltpu.VMEM((1, CE), jnp.int32),
            pltpu.VMEM((1, 3 * N), jnp.float32),
            pltpu.VMEM((1, 3 * N), jnp.float32),
            pltpu.VMEM((1, 3 * N), jnp.float32),
            pltpu.VMEM((3, EP), jnp.float32),
        ],
    )
    return kfn(srcf, dstf, el, er, zeros_n3)


def _deg_body(srcf_hbm, dstf_hbm, zer_hbm, dpart_hbm, src_v, dst_v, dp_v):
    c = lax.axis_index("c")
    s = lax.axis_index("s")
    w = s * NC + c

    pltpu.sync_copy(srcf_hbm.at[w], src_v)
    pltpu.sync_copy(dstf_hbm.at[w], dst_v)
    pltpu.sync_copy(zer_hbm, dp_v)

    ones16 = jnp.ones((16,), jnp.float32)
    z16 = jnp.zeros((16,), jnp.int32)

    def _step(k, _):
        s16 = src_v[0, pl.ds(k * 16, 16)]
        d16 = dst_v[0, pl.ds(k * 16, 16)]
        plsc.addupdate_scatter(dp_v, [z16, s16], ones16)
        plsc.addupdate_scatter(dp_v, [z16, d16 + N], ones16)
        return 0
    lax.fori_loop(0, CE // 16, _step, 0)

    pltpu.sync_copy(dp_v, dpart_hbm.at[w])


def _deg_sc(srcf, dstf, zeros_n2):
    kfn = pl.kernel(
        _deg_body,
        out_type=jax.ShapeDtypeStruct((NW, 1, 2 * N), jnp.float32),
        mesh=_MESH,
        compiler_params=_SC_PARAMS,
        scratch_types=[
            pltpu.VMEM((1, CE), jnp.int32),
            pltpu.VMEM((1, CE), jnp.int32),
            pltpu.VMEM((1, 2 * N), jnp.float32),
        ],
    )
    return kfn(srcf, dstf, zeros_n2)


def _gatagg_body(src2_hbm, dstf_hbm, featp_hbm, ex_hbm,
                 zer_hbm, rst_hbm,
                 idx2_v, dst_v, ex3_v, rbuf0, rbuf1, acc, sem0, sem1):
    c = lax.axis_index("c")
    s = lax.axis_index("s")
    w = s * NC + c

    pltpu.sync_copy(dstf_hbm.at[w], dst_v)
    pltpu.sync_copy(src2_hbm.at[w], idx2_v)

    def _d16(k):
        return dst_v[0, pl.ds(k * 16, 16)]

    cpp = EP // CB

    for h in range(3):
        if h > 0:
            def _mkidx(i, _):
                def _inner(k, _):
                    idx2_v[i, pl.ds(k * 16, 16)] = (
                        idx2_v[i, pl.ds(k * 16, 16)] + N)
                    return 0
                return lax.fori_loop(0, CB // 16, _inner, 0)
            lax.fori_loop(0, NCHUNK, _mkidx, 0)

        pltpu.sync_copy(zer_hbm, acc.at[pl.ds(s * NPT, NPT)])
        plsc.subcore_barrier()

        def _subpass(p, _):
            pltpu.sync_copy(ex_hbm.at[w, p], ex3_v)

            def _scale(buf, cl):
                def _edge(j, _):
                    ev = plsc.load_gather(
                        ex3_v,
                        [jnp.full((16,), h, jnp.int32),
                         jnp.broadcast_to(cl * CB + j, (16,)).astype(jnp.int32)])
                    for v in range(8):
                        buf[j, pl.ds(v * 16, 16)] = buf[j, pl.ds(v * 16, 16)] * ev
                    return 0
                lax.fori_loop(0, CB, _edge, 0)

            def _scatter(buf, cc):
                for q in range(CB // 16):
                    pltpu.sync_copy(buf.at[pl.ds(q * 16, 16)],
                                    acc.at[_d16(cc * (CB // 16) + q)], add=True)

            pltpu.async_copy(featp_hbm.at[idx2_v.at[p * cpp]], rbuf0, sem0)

            def _dstep(g, _):
                c0 = p * cpp + 2 * g
                pltpu.make_async_copy(
                    featp_hbm.at[idx2_v.at[c0]], rbuf0, sem0).wait()
                d1 = pltpu.async_copy(
                    featp_hbm.at[idx2_v.at[c0 + 1]], rbuf1, sem1)
                _scale(rbuf0, 2 * g)
                _scatter(rbuf0, c0)
                d1.wait()
                pltpu.async_copy(featp_hbm.at[idx2_v.at[c0 + 2]], rbuf0, sem0)
                _scale(rbuf1, 2 * g + 1)
                _scatter(rbuf1, c0 + 1)
                return 0
            lax.fori_loop(0, cpp // 2, _dstep, 0)
            ct = p * cpp + cpp - 1
            pltpu.make_async_copy(featp_hbm.at[idx2_v.at[ct]], rbuf0, sem0).wait()
            _scale(rbuf0, cpp - 1)
            _scatter(rbuf0, ct)
            return 0
        lax.fori_loop(0, NEP, _subpass, 0)

        plsc.subcore_barrier()
        pltpu.sync_copy(acc.at[pl.ds(s * NPT, NPT)], rst_hbm.at[h, c, s])


def _gatagg_sc(src2, dstf, featp, ex, zeros_npt):
    kfn = pl.kernel(
        _gatagg_body,
        out_type=jax.ShapeDtypeStruct((3, NC, NS, NPT, 128), jnp.float32),
        mesh=_MESH,
        compiler_params=_SC_PARAMS,
        scratch_types=[
            pltpu.VMEM((NCHUNK, CB), jnp.int32),
            pltpu.VMEM((1, CE), jnp.int32),
            pltpu.VMEM((3, EP), jnp.float32),
            pltpu.VMEM((CB, 128), jnp.float32),
            pltpu.VMEM((CB, 128), jnp.float32),
            pltpu.VMEM_SHARED((N, 128), jnp.float32),
            pltpu.SemaphoreType.DMA,
            pltpu.SemaphoreType.DMA,
        ],
    )
    return kfn(src2, dstf, featp, ex, zeros_npt)


def _att_body(dstf_hbm, ex_hbm, s_hbm, a_hbm, att_hbm,
              dst_v, s_v, ex3_v, ab_v, att_v):
    c = lax.axis_index("c")
    s = lax.axis_index("s")
    w = s * NC + c

    pltpu.sync_copy(dstf_hbm.at[w], dst_v)
    pltpu.sync_copy(s_hbm, s_v)

    z16 = jnp.zeros((16,), jnp.int32)
    for p in range(NEP):
        pltpu.sync_copy(ex_hbm.at[w, p], ex3_v)

        def _att(k, _):
            j = p * EP + k * 16
            d16 = dst_v[0, pl.ds(j, 16)]
            amax = None
            for h2 in range(3):
                a16 = (ex3_v[h2, pl.ds(k * 16, 16)]
                       / plsc.load_gather(s_v, [z16, d16 + h2 * N]))
                ab_v[h2, pl.ds(k * 16, 16)] = a16
                amax = a16 if amax is None else jnp.maximum(amax, a16)
            att_v[0, pl.ds(j, 16)] = amax
            return 0
        lax.fori_loop(0, EP // 16, _att, 0)
        pltpu.sync_copy(ab_v, a_hbm.at[w, p])

    pltpu.sync_copy(att_v, att_hbm.at[w])


def _att_sc(dstf, ex, s_tot):
    kfn = pl.kernel(
        _att_body,
        out_type=[
            jax.ShapeDtypeStruct((NW, NEP, 3, EP), jnp.float32),
            jax.ShapeDtypeStruct((NW, 1, CE), jnp.float32),
        ],
        mesh=_MESH,
        compiler_params=_SC_PARAMS,
        scratch_types=[
            pltpu.VMEM((1, CE), jnp.int32),
            pltpu.VMEM((1, 3 * N), jnp.float32),
            pltpu.VMEM((3, EP), jnp.float32),
            pltpu.VMEM((3, EP), jnp.float32),
            pltpu.VMEM((1, CE), jnp.float32),
        ],
    )
    return kfn(dstf, ex, s_tot)


def _gcnagg_body(src2_hbm, dstf_hbm, tbl_hbm, zer_hbm, out_hbm,
                 src_v, dst_v, rbuf0, rbuf1, acc, sem0, sem1):
    c = lax.axis_index("c")
    s = lax.axis_index("s")
    w = s * NC + c

    pltpu.sync_copy(src2_hbm.at[w], src_v)
    pltpu.sync_copy(dstf_hbm.at[w], dst_v)
    pltpu.sync_copy(zer_hbm, acc.at[pl.ds(s * NPT, NPT)])
    plsc.subcore_barrier()

    def _d16(k):
        return dst_v[0, pl.ds(k * 16, 16)]

    def _scatter(buf, cc):
        for q in range(CB // 16):
            pltpu.sync_copy(buf.at[pl.ds(q * 16, 16)],
                            acc.at[_d16(cc * (CB // 16) + q)], add=True)

    pltpu.async_copy(tbl_hbm.at[src_v.at[0]], rbuf0, sem0)

    def _dstep(g, _):
        c0 = 2 * g
        pltpu.make_async_copy(tbl_hbm.at[src_v.at[c0]], rbuf0, sem0).wait()
        d1 = pltpu.async_copy(tbl_hbm.at[src_v.at[c0 + 1]], rbuf1, sem1)
        _scatter(rbuf0, c0)
        d1.wait()
        pltpu.async_copy(tbl_hbm.at[src_v.at[c0 + 2]], rbuf0, sem0)
        _scatter(rbuf1, c0 + 1)
        return 0
    lax.fori_loop(0, NCHUNK // 2, _dstep, 0)
    ct = NCHUNK - 1
    pltpu.make_async_copy(tbl_hbm.at[src_v.at[ct]], rbuf0, sem0).wait()
    _scatter(rbuf0, ct)

    plsc.subcore_barrier()
    pltpu.sync_copy(acc.at[pl.ds(s * NPT, NPT)], out_hbm.at[c, s])


def _gcnagg_sc(src2, dstf, tbl, zeros_npt):
    kfn = pl.kernel(
        _gcnagg_body,
        out_type=jax.ShapeDtypeStruct((NC, NS, NPT, 128), jnp.float32),
        mesh=_MESH,
        compiler_params=_SC_PARAMS,
        scratch_types=[
            pltpu.VMEM((NCHUNK, CB), jnp.int32),
            pltpu.VMEM((1, CE), jnp.int32),
            pltpu.VMEM((CB, 128), jnp.float32),
            pltpu.VMEM((CB, 128), jnp.float32),
            pltpu.VMEM_SHARED((N, 128), jnp.float32),
            pltpu.SemaphoreType.DMA,
            pltpu.SemaphoreType.DMA,
        ],
    )
    return kfn(src2, dstf, tbl, zeros_npt)


def _pad_gat(W, al, ar, f):
    Wp = W.reshape(W.shape[0], 3, f)
    alp, arp = al, ar
    if f < 128:
        Wp = jnp.pad(Wp, ((0, 0), (0, 0), (0, 128 - f)))
        alp = jnp.pad(al, ((0, 0), (0, 128 - f)))
        arp = jnp.pad(ar, ((0, 0), (0, 128 - f)))
    return Wp.reshape(W.shape[0], 384), alp, arp


def _pad_b(b, f):
    bp = b.reshape(3, f)
    if f < 128:
        bp = jnp.pad(bp, ((0, 0), (0, 128 - f)))
    return bp


def _pad_gcn_w(W):
    if W.shape[0] == 128:
        return W
    return jnp.pad(W, ((0, 128 - W.shape[0]), (0, 0)))


def _graph_pipeline(src, dst, x, Wp, alp, arp, bp, f, gcnA_W, gcnA_b,
                    gcnB_W, gcnB_b, relu_after_max, zn3, zn2, znpt):
    srcf = src.reshape(NW, 1, CE)
    dstf = dst.reshape(NW, 1, CE)
    src2 = src.reshape(NW, NCHUNK, CB)

    featp, el, er = _gat_feat(x, Wp, alp, arp)
    ex, spart = _stats_sc(srcf, dstf, el.reshape(1, 3 * N),
                          er.reshape(1, 3 * N), zn3)
    dpart = _deg_sc(srcf, dstf, zn2)
    s_tot, dfac = _prep(spart.reshape(NW, 3, N), dpart.reshape(NW, 2, N))
    a_planes, att = _att_sc(dstf, ex, s_tot.reshape(1, 3 * N))
    rst_part = _gatagg_sc(src2, dstf, featp.reshape(3 * N, 128), a_planes, znpt)
    rst_part = rst_part.reshape(3, NC, N, 128)

    dfac_t = dfac.T
    y1 = _mid1(rst_part, dfac_t, bp, gcnA_W, relu_after_max)
    agg1 = _gcnagg_sc(src2, dstf, y1, znpt).reshape(NC, N, 128)
    y2 = _mid2(agg1, dfac_t, gcnA_b, gcnB_W)
    agg2 = _gcnagg_sc(src2, dstf, y2, znpt).reshape(NC, N, 128)
    xf = _fin(agg2, dfac_t, gcnB_b)
    return xf, att


def kernel(g1, g2, g3, h1, h2, h3, gat1_W, gat1_al, gat1_ar, gat1_b,
           gat2_W, gat2_al, gat2_ar, gat2_b, c1_W, c1_b, c2_W, c2_b,
           c3_W, c3_b, r1_W, r1_b, r2_W, r2_b, cl_W, cl_b):
    zn3 = jnp.zeros((1, 3 * N), jnp.float32)
    zn2 = jnp.zeros((1, 2 * N), jnp.float32)
    znpt = jnp.zeros((NPT, 128), jnp.float32)

    W1p, al1p, ar1p = _pad_gat(gat1_W, gat1_al, gat1_ar, 127)
    b1p = _pad_b(gat1_b, 127)
    W2p, al2p, ar2p = _pad_gat(gat2_W, gat2_al, gat2_ar, 128)
    b2p = _pad_b(gat2_b, 128)
    c3_Wp = _pad_gcn_w(c3_W)

    x1f, att1 = _graph_pipeline(g1[0], g1[1], h1, W1p, al1p, ar1p, b1p, 127,
                                c3_Wp, c3_b, c2_W, c2_b, True, zn3, zn2, znpt)
    x2f, _ = _graph_pipeline(g2[0], g2[1], h2, W1p, al1p, ar1p, b1p, 127,
                             c3_Wp, c3_b, c2_W, c2_b, False, zn3, zn2, znpt)
    x3f, att3 = _graph_pipeline(g3[0], g3[1], h3, W2p, al2p, ar2p, b2p, 128,
                                c1_W, c1_b, c2_W, c2_b, False, zn3, zn2, znpt)

    reg1, reg2, cls, hg3, wcol = _heads(x1f, x2f, x3f, r1_W, r1_b, r2_W, r2_b,
                                        cl_W, cl_b)
    return (reg1, reg2, cls, att1.reshape(E, 1), att3.reshape(E, 1),
            wcol.reshape(N), hg3)
